# baseline (speedup 1.0000x reference)
import jax
import jax.numpy as jnp
from jax.experimental import pallas as pl

BM = 1024
BMQ = 4096
BATCH = 4096
BETA = 0.25
NSTAGE = 3
K = 256
E = 128


def _enc_body(x_ref, w1, b1, w2, b2, w3, b3, w4, b4, w5, b5, z_ref):
    h = x_ref[...]
    h = jax.nn.relu(jnp.dot(h, w1[...]) + b1[...])
    h = jax.nn.relu(jnp.dot(h, w2[...]) + b2[...])
    h = jax.nn.relu(jnp.dot(h, w3[...]) + b3[...])
    h = jax.nn.relu(jnp.dot(h, w4[...]) + b4[...])
    z_ref[...] = jnp.dot(h, w5[...]) + b5[...]


def _dec_body(q_ref, x_ref, w1, b1, w2, b2, w3, b3, w4, b4, w5, b5,
              out_ref, sq_ref):
    h = q_ref[...]
    h = jax.nn.relu(jnp.dot(h, w1[...]) + b1[...])
    h = jax.nn.relu(jnp.dot(h, w2[...]) + b2[...])
    h = jax.nn.relu(jnp.dot(h, w3[...]) + b3[...])
    h = jax.nn.relu(jnp.dot(h, w4[...]) + b4[...])
    out = jnp.dot(h, w5[...]) + b5[...]
    out_ref[...] = out
    diff = out - x_ref[...]
    part = jnp.sum(diff * diff)

    @pl.when(pl.program_id(0) == 0)
    def _():
        sq_ref[...] = jnp.zeros_like(sq_ref)

    sq_ref[...] += part[None, None]


def _rq_body(z_ref, cb1, cb2, cb3, q_ref, idx_ref, loss_ref):
    r = z_ref[...]
    xq_acc = jnp.zeros_like(r)
    idx_list = []
    loss_parts = []
    for cb_ref in (cb1, cb2, cb3):
        cb = cb_ref[...]
        a = jnp.sum(r ** 2, axis=1, keepdims=True)
        b = jnp.sum(cb ** 2, axis=1)
        c = jax.lax.dot_general(r, cb, (((1,), (1,)), ((), ())))
        d = (a + b[None, :]) - 2.0 * c
        dmin = jnp.min(d, axis=1, keepdims=True)
        iota = jax.lax.broadcasted_iota(jnp.int32, d.shape, 1)
        cand = jnp.where(d == dmin, iota, K)
        idx = jnp.min(cand, axis=1)
        onehot = (iota == idx[:, None]).astype(jnp.float32)
        xq = jax.lax.dot_general(onehot, cb, (((1,), (0,)), ((), ())),
                                 precision=jax.lax.Precision.HIGHEST)
        diff = xq - r
        xq_st = r + diff
        loss_parts.append(jnp.sum(diff * diff))
        idx_list.append(idx)
        xq_acc = xq_acc + xq_st
        r = r - xq_st
    q_ref[...] = xq_acc
    idx_ref[...] = jnp.stack(idx_list, axis=-1)

    @pl.when(pl.program_id(0) == 0)
    def _():
        loss_ref[...] = jnp.zeros_like(loss_ref)

    loss_ref[...] += jnp.stack(loss_parts)[None, :]


def _full(shape):
    return pl.BlockSpec(shape, lambda i: tuple(0 for _ in shape))


def _mlp_specs(Ws):
    specs = []
    for w in Ws:
        specs.append(_full(w.shape))
        specs.append(_full((1, w.shape[1])))
    return specs


def _encoder(x, Ws, bs):
    bm = 2048
    grid = (BATCH // bm,)
    in_specs = [pl.BlockSpec((bm, Ws[0].shape[0]), lambda i: (i, 0))]
    in_specs += _mlp_specs(Ws)
    args = [x]
    for w, b in zip(Ws, bs):
        args += [w, b.reshape(1, -1)]
    return pl.pallas_call(
        _enc_body,
        grid=grid,
        in_specs=in_specs,
        out_specs=pl.BlockSpec((bm, Ws[-1].shape[1]), lambda i: (i, 0)),
        out_shape=jax.ShapeDtypeStruct((BATCH, Ws[-1].shape[1]), jnp.float32),
    )(*args)


def _decoder(q, x_orig, Ws, bs):
    grid = (BATCH // BM,)
    in_specs = [pl.BlockSpec((BM, Ws[0].shape[0]), lambda i: (i, 0)),
                pl.BlockSpec((BM, Ws[-1].shape[1]), lambda i: (i, 0))]
    in_specs += _mlp_specs(Ws)
    args = [q, x_orig]
    for w, b in zip(Ws, bs):
        args += [w, b.reshape(1, -1)]
    out, sq = pl.pallas_call(
        _dec_body,
        grid=grid,
        in_specs=in_specs,
        out_specs=[pl.BlockSpec((BM, Ws[-1].shape[1]), lambda i: (i, 0)),
                   pl.BlockSpec((1, 1), lambda i: (0, 0))],
        out_shape=[jax.ShapeDtypeStruct((BATCH, Ws[-1].shape[1]), jnp.float32),
                   jax.ShapeDtypeStruct((1, 1), jnp.float32)],
    )(*args)
    return out, sq


def _rq(z, cbs):
    grid = (BATCH // BMQ,)
    in_specs = [pl.BlockSpec((BMQ, E), lambda i: (i, 0)),
                _full((K, E)), _full((K, E)), _full((K, E))]
    return pl.pallas_call(
        _rq_body,
        grid=grid,
        in_specs=in_specs,
        out_specs=[pl.BlockSpec((BMQ, E), lambda i: (i, 0)),
                   pl.BlockSpec((BMQ, NSTAGE), lambda i: (i, 0)),
                   pl.BlockSpec((1, NSTAGE), lambda i: (0, 0))],
        out_shape=[jax.ShapeDtypeStruct((BATCH, E), jnp.float32),
                   jax.ShapeDtypeStruct((BATCH, NSTAGE), jnp.int32),
                   jax.ShapeDtypeStruct((1, NSTAGE), jnp.float32)],
    )(z, cbs[0], cbs[1], cbs[2])


def _rql(loss_sums):
    m = loss_sums[0] / float(BATCH * E)
    per_stage = m + BETA * m
    return (per_stage[0] + per_stage[1] + per_stage[2]) / 3.0


def kernel(x_text, x_image, params):
    z_t = _encoder(x_text, params['enc_t'][0], params['enc_t'][1])
    q_t, idx_t, loss_t = _rq(z_t, params['cb_t'])
    out_t, sq_t = _decoder(q_t, x_text, params['dec_t'][0], params['dec_t'][1])

    z_i = _encoder(x_image, params['enc_i'][0], params['enc_i'][1])
    q_i, idx_i, loss_i = _rq(z_i, params['cb_i'])
    out_i, sq_i = _decoder(q_i, x_image, params['dec_i'][0], params['dec_i'][1])

    nrec = float(BATCH * 1024)
    recon = sq_t[0, 0] / nrec + sq_i[0, 0] / nrec
    total = recon + (_rql(loss_t) + _rql(loss_i))
    return out_t, out_i, total, idx_t, idx_i


# speedup vs baseline: 1.0568x; 1.0568x over previous
import jax
import jax.numpy as jnp
from jax.experimental import pallas as pl

BM = 1024
BMQ = 4096
BATCH = 4096
BETA = 0.25
NSTAGE = 3
K = 256
E = 128


def _enc_body(x_ref, w1, b1, w2, b2, w3, b3, w4, b4, w5, b5, z_ref):
    h = x_ref[...]
    h = jax.nn.relu(jnp.dot(h, w1[...]) + b1[...])
    h = jax.nn.relu(jnp.dot(h, w2[...]) + b2[...])
    h = jax.nn.relu(jnp.dot(h, w3[...]) + b3[...])
    h = jax.nn.relu(jnp.dot(h, w4[...]) + b4[...])
    z_ref[...] = jnp.dot(h, w5[...]) + b5[...]


def _dec_body(q_ref, x_ref, w1, b1, w2, b2, w3, b3, w4, b4, w5, b5,
              out_ref, sq_ref):
    h = q_ref[...]
    h = jax.nn.relu(jnp.dot(h, w1[...]) + b1[...])
    h = jax.nn.relu(jnp.dot(h, w2[...]) + b2[...])
    h = jax.nn.relu(jnp.dot(h, w3[...]) + b3[...])
    h = jax.nn.relu(jnp.dot(h, w4[...]) + b4[...])
    out = jnp.dot(h, w5[...]) + b5[...]
    out_ref[...] = out
    diff = out - x_ref[...]
    part = jnp.sum(diff * diff)

    @pl.when(pl.program_id(0) == 0)
    def _():
        sq_ref[...] = jnp.zeros_like(sq_ref)

    sq_ref[...] += part[None, None]


def _rq_body(z_ref, cb1, cb2, cb3, q_ref, idx_ref, loss_ref):
    r = z_ref[...]
    xq_acc = jnp.zeros_like(r)
    idx_list = []
    loss_parts = []
    for cb_ref in (cb1, cb2, cb3):
        cb = cb_ref[...]
        a = jnp.sum(r ** 2, axis=1, keepdims=True)
        b = jnp.sum(cb ** 2, axis=1)
        c = jax.lax.dot_general(r, cb, (((1,), (1,)), ((), ())))
        d = (a + b[None, :]) - 2.0 * c
        dmin = jnp.min(d, axis=1, keepdims=True)
        iota = jax.lax.broadcasted_iota(jnp.int32, d.shape, 1)
        cand = jnp.where(d == dmin, iota, K)
        idx = jnp.min(cand, axis=1)
        onehot = (iota == idx[:, None]).astype(jnp.float32)
        xq = jax.lax.dot_general(onehot, cb, (((1,), (0,)), ((), ())),
                                 precision=jax.lax.Precision.HIGHEST)
        diff = xq - r
        xq_st = r + diff
        loss_parts.append(jnp.sum(diff * diff))
        idx_list.append(idx)
        xq_acc = xq_acc + xq_st
        r = r - xq_st
    q_ref[...] = xq_acc
    idx_ref[...] = jnp.stack(idx_list, axis=-1)

    @pl.when(pl.program_id(0) == 0)
    def _():
        loss_ref[...] = jnp.zeros_like(loss_ref)

    loss_ref[...] += jnp.stack(loss_parts)[None, :]


def _full(shape):
    return pl.BlockSpec(shape, lambda i: tuple(0 for _ in shape))


def _mlp_specs(Ws):
    specs = []
    for w in Ws:
        specs.append(_full(w.shape))
        specs.append(_full((1, w.shape[1])))
    return specs


def _encoder(x, Ws, bs):
    bm = BM
    grid = (BATCH // bm,)
    in_specs = [pl.BlockSpec((bm, Ws[0].shape[0]), lambda i: (i, 0))]
    in_specs += _mlp_specs(Ws)
    args = [x]
    for w, b in zip(Ws, bs):
        args += [w, b.reshape(1, -1)]
    return pl.pallas_call(
        _enc_body,
        grid=grid,
        in_specs=in_specs,
        out_specs=pl.BlockSpec((bm, Ws[-1].shape[1]), lambda i: (i, 0)),
        out_shape=jax.ShapeDtypeStruct((BATCH, Ws[-1].shape[1]), jnp.float32),
    )(*args)


def _decoder(q, x_orig, Ws, bs):
    grid = (BATCH // BM,)
    in_specs = [pl.BlockSpec((BM, Ws[0].shape[0]), lambda i: (i, 0)),
                pl.BlockSpec((BM, Ws[-1].shape[1]), lambda i: (i, 0))]
    in_specs += _mlp_specs(Ws)
    args = [q, x_orig]
    for w, b in zip(Ws, bs):
        args += [w, b.reshape(1, -1)]
    out, sq = pl.pallas_call(
        _dec_body,
        grid=grid,
        in_specs=in_specs,
        out_specs=[pl.BlockSpec((BM, Ws[-1].shape[1]), lambda i: (i, 0)),
                   pl.BlockSpec((1, 1), lambda i: (0, 0))],
        out_shape=[jax.ShapeDtypeStruct((BATCH, Ws[-1].shape[1]), jnp.float32),
                   jax.ShapeDtypeStruct((1, 1), jnp.float32)],
    )(*args)
    return out, sq


def _rq(z, cbs):
    grid = (BATCH // BMQ,)
    in_specs = [pl.BlockSpec((BMQ, E), lambda i: (i, 0)),
                _full((K, E)), _full((K, E)), _full((K, E))]
    return pl.pallas_call(
        _rq_body,
        grid=grid,
        in_specs=in_specs,
        out_specs=[pl.BlockSpec((BMQ, E), lambda i: (i, 0)),
                   pl.BlockSpec((BMQ, NSTAGE), lambda i: (i, 0)),
                   pl.BlockSpec((1, NSTAGE), lambda i: (0, 0))],
        out_shape=[jax.ShapeDtypeStruct((BATCH, E), jnp.float32),
                   jax.ShapeDtypeStruct((BATCH, NSTAGE), jnp.int32),
                   jax.ShapeDtypeStruct((1, NSTAGE), jnp.float32)],
    )(z, cbs[0], cbs[1], cbs[2])


def _rql(loss_sums):
    m = loss_sums[0] / float(BATCH * E)
    per_stage = m + BETA * m
    return (per_stage[0] + per_stage[1] + per_stage[2]) / 3.0


def kernel(x_text, x_image, params):
    z_t = _encoder(x_text, params['enc_t'][0], params['enc_t'][1])
    q_t, idx_t, loss_t = _rq(z_t, params['cb_t'])
    out_t, sq_t = _decoder(q_t, x_text, params['dec_t'][0], params['dec_t'][1])

    z_i = _encoder(x_image, params['enc_i'][0], params['enc_i'][1])
    q_i, idx_i, loss_i = _rq(z_i, params['cb_i'])
    out_i, sq_i = _decoder(q_i, x_image, params['dec_i'][0], params['dec_i'][1])

    nrec = float(BATCH * 1024)
    recon = sq_t[0, 0] / nrec + sq_i[0, 0] / nrec
    total = recon + (_rql(loss_t) + _rql(loss_i))
    return out_t, out_i, total, idx_t, idx_i
